# baseline (speedup 1.0000x reference)
import functools

import jax
import jax.numpy as jnp
from jax import lax
from jax.experimental import pallas as pl
from jax.experimental.pallas import tpu as pltpu
from jax.experimental.pallas import tpu_sc as plsc

_B = 16384
_V = 1000
_D = 1000
_DA = 896
_DT = _D - _DA
_DTP = 128

_info = plsc.get_sparse_core_info()
_NC = _info.num_cores
_NS = _info.num_subcores
_NW = _NC * _NS
_CH = 32
_LANES = _info.num_lanes

_mesh = plsc.VectorSubcoreMesh(core_axis_name="c", subcore_axis_name="s")


def _make_gather(rows, out_rows):
    bpw = rows // _NW
    nch = bpw // _CH

    @functools.partial(
        pl.kernel,
        mesh=_mesh,
        out_type=jax.ShapeDtypeStruct((out_rows, _D), jnp.float32),
        scratch_types=[
            pltpu.VMEM((bpw,), jnp.int32),
            pltpu.VMEM((_CH, _D), jnp.float32),
            pltpu.VMEM((_CH, _D), jnp.float32),
            pltpu.VMEM((_CH, _DTP), jnp.float32),
            pltpu.VMEM((_CH, _DTP), jnp.float32),
            pltpu.SemaphoreType.DMA,
            pltpu.SemaphoreType.DMA,
            pltpu.SemaphoreType.DMA,
            pltpu.SemaphoreType.DMA,
            pltpu.SemaphoreType.DMA,
            pltpu.SemaphoreType.DMA,
        ],
        compiler_params=pltpu.CompilerParams(needs_layout_passes=False),
    )
    def _gather(y_hbm, ta_hbm, tb_hbm, out_hbm, idx_v, buf0, buf1, tail0,
                tail1, ga0, ga1, gb0, gb1, ss0, ss1):
        wid = lax.axis_index("s") * _NC + lax.axis_index("c")
        base = wid * bpw
        pltpu.sync_copy(y_hbm.at[pl.ds(base, bpw)], idx_v)
        for i in range(bpw // _LANES):
            sl = pl.ds(i * _LANES, _LANES)
            v = idx_v[sl]
            idx_v[sl] = jnp.minimum(jnp.maximum(v, 0), _V - 1)

        lane = lax.iota(jnp.int32, _LANES)
        lo8 = lane < 8
        tail_cols = (_DA + 6 * _LANES) + (lane & 7)

        def _fixup(buf, tail):
            def row(r, _):
                for k in range(_DT // _LANES):
                    tv = tail[r, pl.ds(k * _LANES, _LANES)]
                    buf[r, pl.ds(_DA + k * _LANES, _LANES)] = tv
                v = tail[r, pl.ds(96, _LANES)]
                rws = jnp.full((_LANES,), r, jnp.int32)
                plsc.store_scatter(buf, [rws, tail_cols], v, mask=lo8)
                return _
            lax.fori_loop(0, _CH, row, 0)

        bufs = (buf0, buf1)
        tails = (tail0, tail1)
        gasems = (ga0, ga1)
        gbsems = (gb0, gb1)
        ssems = (ss0, ss1)
        gaths = [None, None]
        stores = [None, None]
        for c in range(nch):
            s = c % 2
            if stores[s] is not None:
                stores[s].wait()
            isl = idx_v.at[pl.ds(c * _CH, _CH)]
            gaths[s] = (
                pltpu.async_copy(ta_hbm.at[isl], bufs[s].at[:, pl.ds(0, _DA)],
                                 gasems[s]),
                pltpu.async_copy(tb_hbm.at[isl], tails[s], gbsems[s]),
            )
            if c >= 1:
                p = (c - 1) % 2
                gaths[p][0].wait()
                gaths[p][1].wait()
                _fixup(bufs[p], tails[p])
                stores[p] = pltpu.async_copy(
                    bufs[p], out_hbm.at[pl.ds(base + (c - 1) * _CH, _CH)],
                    ssems[p])
        last = (nch - 1) % 2
        gaths[last][0].wait()
        gaths[last][1].wait()
        _fixup(bufs[last], tails[last])
        stores[last] = pltpu.async_copy(
            bufs[last], out_hbm.at[pl.ds(base + (nch - 1) * _CH, _CH)],
            ssems[last])
        stores[1 - last].wait()
        stores[last].wait()

    return _gather


_R0 = 12288
_R1 = _B - _R0
_gather0 = _make_gather(_R0, _B)
_gather1 = _make_gather(_R1, _R1)


def kernel(y_n, unique_cell_types, ancestors):
    del unique_cell_types
    table_a = ancestors[:, :_DA]
    table_b = jnp.pad(ancestors[:, _DA:], ((0, 0), (0, _DTP - _DT)))
    out = _gather0(y_n[:_R0], table_a, table_b)
    out1 = _gather1(y_n[_R0:], table_a, table_b)
    return lax.dynamic_update_slice(out, out1, (_R0, 0))
